# baseline (speedup 1.0000x reference)
import functools

import jax
import jax.numpy as jnp
from jax import lax
from jax.experimental import pallas as pl
from jax.experimental.pallas import tpu as pltpu
from jax.experimental.pallas import tpu_sc as plsc

D = 128
W = D // 2
CPR = 128
NBUF = 4


@functools.partial(jax.jit, static_argnames=("nw", "chunks"))
def _gather_rows(tbl_words, idx2d, *, nw, chunks):
    rows_per_w = chunks * CPR
    mesh = plsc.VectorSubcoreMesh(core_axis_name="c", subcore_axis_name="s")
    nc = mesh.num_cores

    @functools.partial(
        pl.kernel,
        out_type=jax.ShapeDtypeStruct((nw * rows_per_w, W), jnp.int32),
        mesh=mesh,
        compiler_params=pltpu.CompilerParams(use_tc_tiling_on_sc=False),
        scratch_types=[
            pltpu.VMEM((chunks, CPR), jnp.int32),
            pltpu.VMEM((NBUF, CPR, W), jnp.int32),
            pltpu.VMEM((NBUF, CPR, W), jnp.int32),
            pltpu.SemaphoreType.DMA((NBUF,)),
            pltpu.SemaphoreType.DMA((NBUF,)),
        ],
    )
    def k(tbl_hbm, idx_hbm, out_hbm, idx_v, wbuf, obuf, gsem, ssem):
        wid = lax.axis_index("s") * nc + lax.axis_index("c")
        row0 = wid * rows_per_w
        pltpu.sync_copy(idx_hbm.at[pl.ds(wid * chunks, chunks)], idx_v)

        hi_mask = jnp.full((16,), jnp.int32(-65536))

        def widen(b):
            @pl.loop(0, CPR, unroll=8)
            def _(r):
                for kblk in range(W // 16):
                    w = wbuf[b, r, pl.ds(kblk * 16, 16)]
                    obuf[b, r, pl.ds(kblk * 16, 16)] = w << 16

        for b in range(NBUF):
            pltpu.async_copy(tbl_hbm.at[idx_v.at[b]], wbuf.at[b], gsem.at[b])

        @pl.loop(0, chunks - NBUF, step=NBUF)
        def _(j0):
            for b in range(NBUF):
                j = j0 + b
                dst = out_hbm.at[pl.ds(row0 + j * CPR, CPR)]
                pltpu.make_async_copy(
                    tbl_hbm.at[idx_v.at[j]], wbuf.at[b], gsem.at[b]
                ).wait()

                @pl.when(j0 > 0)
                def _():
                    pltpu.make_async_copy(
                        obuf.at[b],
                        out_hbm.at[pl.ds(row0 + (j - NBUF) * CPR, CPR)],
                        ssem.at[b],
                    ).wait()

                widen(b)
                pltpu.async_copy(obuf.at[b], dst, ssem.at[b])
                pltpu.async_copy(
                    tbl_hbm.at[idx_v.at[j + NBUF]], wbuf.at[b], gsem.at[b]
                )

        for b in range(NBUF):
            j = chunks - NBUF + b
            dst = out_hbm.at[pl.ds(row0 + j * CPR, CPR)]
            pltpu.make_async_copy(
                tbl_hbm.at[idx_v.at[j]], wbuf.at[b], gsem.at[b]
            ).wait()
            pltpu.make_async_copy(
                obuf.at[b],
                out_hbm.at[pl.ds(row0 + (j - NBUF) * CPR, CPR)],
                ssem.at[b],
            ).wait()
            widen(b)
            pltpu.async_copy(obuf.at[b], dst, ssem.at[b])

        for b in range(NBUF):
            j = chunks - NBUF + b
            pltpu.make_async_copy(
                obuf.at[b],
                out_hbm.at[pl.ds(row0 + j * CPR, CPR)],
                ssem.at[b],
            ).wait()

    return k(tbl_words, idx2d)


def kernel(timesteps, pe):
    bsz, hist = timesteps.shape
    total = bsz * hist
    nw = 32
    chunks = total // (nw * CPR)
    assert chunks * nw * CPR == total
    k32 = jnp.arange(0, D, 32)[:, None]
    i16 = jnp.arange(16)[None, :]
    lo_src = (k32 + i16).reshape(-1, 16)
    hi_src = (k32 + 16 + i16).reshape(-1, 16)
    perm = jnp.stack([lo_src, hi_src], axis=2).reshape(D)
    tbl_words = jnp.zeros((pe.shape[0], W), jnp.int32) + timesteps[0, 0]
    idx2d = timesteps.reshape(nw * chunks, CPR)
    out = _gather_rows(tbl_words, idx2d, nw=nw, chunks=chunks)
    out = jax.lax.bitcast_convert_type(out, jnp.float32)
    out = jnp.repeat(out, 2, axis=1)
    return out.reshape(bsz, hist, pe.shape[1])


# speedup vs baseline: 8.9048x; 8.9048x over previous
import functools

import jax
import jax.numpy as jnp
from jax import lax
from jax.experimental import pallas as pl
from jax.experimental.pallas import tpu as pltpu
from jax.experimental.pallas import tpu_sc as plsc

D = 128
CPR = 128
NBUF = 4


@functools.partial(jax.jit, static_argnames=("nw", "chunks"))
def _gather_rows(pe, idx2d, *, nw, chunks):
    rows_per_w = chunks * CPR
    mesh = plsc.VectorSubcoreMesh(core_axis_name="c", subcore_axis_name="s")
    nc = mesh.num_cores

    @functools.partial(
        pl.kernel,
        out_type=jax.ShapeDtypeStruct((nw * rows_per_w, D), jnp.float32),
        mesh=mesh,
        scratch_types=[
            pltpu.VMEM((chunks, CPR), jnp.int32),
            pltpu.VMEM((NBUF, CPR, D), jnp.float32),
            pltpu.SemaphoreType.DMA((NBUF,)),
            pltpu.SemaphoreType.DMA((NBUF,)),
        ],
    )
    def k(pe_hbm, idx_hbm, out_hbm, idx_v, rows, gsem, ssem):
        wid = lax.axis_index("s") * nc + lax.axis_index("c")
        row0 = wid * rows_per_w
        pltpu.sync_copy(idx_hbm.at[pl.ds(wid * chunks, chunks)], idx_v)

        for b in range(NBUF):
            pltpu.async_copy(pe_hbm.at[idx_v.at[b]], rows.at[b], gsem.at[b])

        @pl.loop(0, chunks - NBUF, step=NBUF)
        def _(j0):
            for b in range(NBUF):
                j = j0 + b
                dst = out_hbm.at[pl.ds(row0 + j * CPR, CPR)]
                pltpu.make_async_copy(
                    pe_hbm.at[idx_v.at[j]], rows.at[b], gsem.at[b]
                ).wait()
                pltpu.async_copy(rows.at[b], dst, ssem.at[b])
                pltpu.make_async_copy(rows.at[b], dst, ssem.at[b]).wait()
                pltpu.async_copy(
                    pe_hbm.at[idx_v.at[j + NBUF]], rows.at[b], gsem.at[b]
                )

        for b in range(NBUF):
            j = chunks - NBUF + b
            dst = out_hbm.at[pl.ds(row0 + j * CPR, CPR)]
            pltpu.make_async_copy(
                pe_hbm.at[idx_v.at[j]], rows.at[b], gsem.at[b]
            ).wait()
            pltpu.async_copy(rows.at[b], dst, ssem.at[b])
            pltpu.make_async_copy(rows.at[b], dst, ssem.at[b]).wait()

    return k(pe, idx2d)


def kernel(timesteps, pe):
    bsz, hist = timesteps.shape
    total = bsz * hist
    nw = 32
    chunks = total // (nw * CPR)
    assert chunks * nw * CPR == total
    idx2d = timesteps.reshape(nw * chunks, CPR)
    out = _gather_rows(pe, idx2d, nw=nw, chunks=chunks)
    return out.reshape(bsz, hist, pe.shape[1])


# speedup vs baseline: 8.9076x; 1.0003x over previous
import functools

import jax
import jax.numpy as jnp
from jax import lax
from jax.experimental import pallas as pl
from jax.experimental.pallas import tpu as pltpu
from jax.experimental.pallas import tpu_sc as plsc

D = 128
CPR = 128
NBUF = 4


@functools.partial(jax.jit, static_argnames=("nw", "chunks"))
def _gather_rows(pe, idx2d, *, nw, chunks):
    rows_per_w = chunks * CPR
    mesh = plsc.VectorSubcoreMesh(core_axis_name="c", subcore_axis_name="s")
    nc = mesh.num_cores

    @functools.partial(
        pl.kernel,
        out_type=jax.ShapeDtypeStruct((nw * rows_per_w, D), jnp.float32),
        mesh=mesh,
        compiler_params=pltpu.CompilerParams(disable_bounds_checks=True),
        scratch_types=[
            pltpu.VMEM((chunks, CPR), jnp.int32),
            pltpu.VMEM((NBUF, CPR, D), jnp.float32),
            pltpu.SemaphoreType.DMA((NBUF,)),
            pltpu.SemaphoreType.DMA((NBUF,)),
        ],
    )
    def k(pe_hbm, idx_hbm, out_hbm, idx_v, rows, gsem, ssem):
        wid = lax.axis_index("s") * nc + lax.axis_index("c")
        row0 = wid * rows_per_w
        pltpu.sync_copy(idx_hbm.at[pl.ds(wid * chunks, chunks)], idx_v)

        for b in range(NBUF):
            pltpu.async_copy(pe_hbm.at[idx_v.at[b]], rows.at[b], gsem.at[b])

        @pl.loop(0, chunks - NBUF, step=NBUF)
        def _(j0):
            for b in range(NBUF):
                j = j0 + b
                dst = out_hbm.at[pl.ds(row0 + j * CPR, CPR)]
                pltpu.make_async_copy(
                    pe_hbm.at[idx_v.at[j]], rows.at[b], gsem.at[b]
                ).wait()
                pltpu.async_copy(rows.at[b], dst, ssem.at[b])
                pltpu.make_async_copy(rows.at[b], dst, ssem.at[b]).wait()
                pltpu.async_copy(
                    pe_hbm.at[idx_v.at[j + NBUF]], rows.at[b], gsem.at[b]
                )

        for b in range(NBUF):
            j = chunks - NBUF + b
            dst = out_hbm.at[pl.ds(row0 + j * CPR, CPR)]
            pltpu.make_async_copy(
                pe_hbm.at[idx_v.at[j]], rows.at[b], gsem.at[b]
            ).wait()
            pltpu.async_copy(rows.at[b], dst, ssem.at[b])
            pltpu.make_async_copy(rows.at[b], dst, ssem.at[b]).wait()

    return k(pe, idx2d)


def kernel(timesteps, pe):
    bsz, hist = timesteps.shape
    total = bsz * hist
    nw = 32
    chunks = total // (nw * CPR)
    assert chunks * nw * CPR == total
    idx2d = timesteps.reshape(nw * chunks, CPR)
    out = _gather_rows(pe, idx2d, nw=nw, chunks=chunks)
    return out.reshape(bsz, hist, pe.shape[1])
